# baseline (speedup 1.0000x reference)
import functools
import jax
import jax.numpy as jnp
from jax import lax
from jax.experimental import pallas as pl
from jax.experimental.pallas import tpu as pltpu
from jax.experimental.pallas import tpu_sc as plsc

_B = 512
_L = 16
_P = 240
_H = 768
_S = _L * _L
_PB = 16

_BT = 384
_BS = _B - _BT
_NW = 32
_PW = _BS // _NW
_CH = 8
_NBUF = 4
_RG = 8
_NJ = _H // 16


def _tc_body(x_ref, p0_ref, p1_ref, w_ref, b_ref, eps_ref, out_ref):
    x = x_ref[...]
    w = w_ref[...]
    eps = eps_ref[0, 0]
    bias = b_ref[0, 0]
    logits = jnp.sum(x * w[None, :, :], axis=2, keepdims=True) + bias
    val = jax.nn.sigmoid(logits)
    idx = p0_ref[...] * _L + p1_ref[...]
    k3 = lax.broadcasted_iota(jnp.int32, (_PB, _P, 1), 1).astype(jnp.float32)
    packed = k3 + val
    lane = lax.broadcasted_iota(jnp.int32, (_PB, _P, _S), 2)
    comb = jnp.where(idx == lane, packed, -1.0)
    red = jnp.max(comb, axis=1)
    frac = red - jnp.floor(red)
    out_ref[...] = jnp.where(red >= 0.0, frac, 1.0) + eps


def _tc_part(x3, p0, p1, W, b2, eps2):
    grid = (_BT // _PB,)
    return pl.pallas_call(
        _tc_body,
        grid=grid,
        in_specs=[
            pl.BlockSpec((_PB, _P, _H), lambda i: (i, 0, 0)),
            pl.BlockSpec((_PB, _P, 1), lambda i: (i, 0, 0)),
            pl.BlockSpec((_PB, _P, 1), lambda i: (i, 0, 0)),
            pl.BlockSpec((1, _H), lambda i: (0, 0)),
            pl.BlockSpec((1, 1), lambda i: (0, 0)),
            pl.BlockSpec((1, 1), lambda i: (0, 0)),
        ],
        out_specs=pl.BlockSpec((_PB, _S), lambda i: (i, 0)),
        out_shape=jax.ShapeDtypeStruct((_BT, _S), jnp.float32),
    )(x3, p0, p1, W, b2, eps2)


_ROWS_W = _PW * _P
_NCHUNK = _ROWS_W // _CH

_LANE_IOTA = None


def _sc_kernel_body(x_hbm, p0_hbm, p1_hbm, w_hbm, be_hbm, out_hbm,
                    xb0, xb1, xb2, xb3, wv, p0v, p1v, logv, adjv, bev,
                    sem0, sem1, sem2, sem3):
    nc = 2
    wid = lax.axis_index("s") * nc + lax.axis_index("c")
    row0 = (_BT + wid * _PW) * _P
    xoff0 = row0 * _H

    pltpu.sync_copy(w_hbm, wv)
    pltpu.sync_copy(be_hbm, bev)
    pltpu.sync_copy(p0_hbm.at[pl.ds(row0, _ROWS_W)], p0v)
    pltpu.sync_copy(p1_hbm.at[pl.ds(row0, _ROWS_W)], p1v)

    cbytes = _CH * _H
    bufs = [xb0, xb1, xb2, xb3]
    sems = [sem0, sem1, sem2, sem3]

    def _start(b, chunk):
        pltpu.make_async_copy(
            x_hbm.at[pl.ds(xoff0 + chunk * cbytes, cbytes)], bufs[b],
            sems[b]).start()

    def _wait(b):
        pltpu.make_async_copy(
            x_hbm.at[pl.ds(0, cbytes)], bufs[b], sems[b]).wait()

    lanes0 = lax.iota(jnp.int32, 16)
    gdims0 = lax.GatherDimensionNumbers(
        offset_dims=(), collapsed_slice_dims=(0,), start_index_map=(0,))
    xor_idx = [(lanes0 ^ m).reshape(16, 1) for m in (8, 4, 2, 1)]

    def _allsum(v):
        for ix in xor_idx:
            v = v + lax.gather(v, ix, gdims0, (1,),
                               mode=lax.GatherScatterMode.PROMISE_IN_BOUNDS)
        return v

    def _compute_chunk(buf, chunk):
        out = jnp.zeros((16,), jnp.float32)
        for rg in range(_CH // _RG):
            accs = [jnp.zeros((16,), jnp.float32) for _ in range(_RG)]
            for j in range(_NJ):
                wj = wv[pl.ds(16 * j, 16)]
                for r in range(_RG):
                    xo = (rg * _RG + r) * _H + 16 * j
                    accs[r] = accs[r] + buf[pl.ds(xo, 16)] * wj
            for r in range(_RG):
                row = rg * _RG + r
                out = jnp.where(lanes0 == row, _allsum(accs[r]), out)
        logv[pl.ds(chunk * _CH, 16)] = out

    for b in range(_NBUF):
        _start(b, b)

    def _round(c, carry):
        for b in range(_NBUF):
            chunk = c * _NBUF + b
            _wait(b)
            _compute_chunk(bufs[b], chunk)

            @pl.when(chunk + _NBUF < _NCHUNK)
            def _():
                _start(b, chunk + _NBUF)
        return carry

    lax.fori_loop(0, _NCHUNK // _NBUF, _round, 0)

    bvec = bev[pl.ds(0, 16)]
    evec = bev[pl.ds(16, 16)]
    ones_eps = 1.0 + evec

    def _init(i, carry):
        adjv[pl.ds(i * 16, 16)] = ones_eps
        return carry

    lax.fori_loop(0, _PW * _S // 16, _init, 0)

    lanes = lax.iota(jnp.int32, 16)
    shifted = [jnp.minimum(lanes + sh, 15).reshape(16, 1) for sh in range(1, 16)]
    later_ok = [lanes + sh <= 15 for sh in range(1, 16)]
    gdims = lax.GatherDimensionNumbers(
        offset_dims=(), collapsed_slice_dims=(0,), start_index_map=(0,))

    def _permute(vec, idx):
        return lax.gather(vec, idx, gdims, (1,),
                          mode=lax.GatherScatterMode.PROMISE_IN_BOUNDS)

    def _group(g, carry):
        off = g * 16
        lg = logv[pl.ds(off, 16)] + bvec
        vals = 1.0 / (1.0 + jnp.exp(-lg)) + evec
        slot = p0v[pl.ds(off, 16)] * _L + p1v[pl.ds(off, 16)]
        keep = lanes >= 0
        for sh in range(15):
            nb = _permute(slot, shifted[sh])
            keep = jnp.logical_and(
                keep, jnp.logical_or(nb != slot, jnp.logical_not(later_ok[sh])))
        sabs = slot + (g // 15) * _S
        plsc.store_scatter(adjv, [sabs], vals, mask=keep)
        return carry

    lax.fori_loop(0, _ROWS_W // 16, _group, 0)

    pltpu.sync_copy(adjv, out_hbm.at[pl.ds(wid * _PW * _S, _PW * _S)])


def _sc_part(x_flat, p0_flat, p1_flat, w_flat, be):
    mesh = plsc.VectorSubcoreMesh(core_axis_name="c", subcore_axis_name="s")
    k = pl.kernel(
        _sc_kernel_body,
        mesh=mesh,
        compiler_params=pltpu.CompilerParams(needs_layout_passes=False),
        out_type=jax.ShapeDtypeStruct((_BS * _S,), jnp.float32),
        scratch_types=[
            pltpu.VMEM((_CH * _H,), jnp.float32),
            pltpu.VMEM((_CH * _H,), jnp.float32),
            pltpu.VMEM((_CH * _H,), jnp.float32),
            pltpu.VMEM((_CH * _H,), jnp.float32),
            pltpu.VMEM((_H,), jnp.float32),
            pltpu.VMEM((_ROWS_W,), jnp.int32),
            pltpu.VMEM((_ROWS_W,), jnp.int32),
            pltpu.VMEM((_ROWS_W + 8,), jnp.float32),
            pltpu.VMEM((_PW * _S,), jnp.float32),
            pltpu.VMEM((32,), jnp.float32),
            pltpu.SemaphoreType.DMA,
            pltpu.SemaphoreType.DMA,
            pltpu.SemaphoreType.DMA,
            pltpu.SemaphoreType.DMA,
        ],
    )
    return k(x_flat, p0_flat, p1_flat, w_flat, be)


def kernel(pooled_output, pairs_list, passage_length, pairs_num, W, b, epsilon):
    del passage_length, pairs_num
    x3 = pooled_output.reshape(_B, _P, _H)
    p0 = pairs_list[:, 0].reshape(_B, _P, 1)
    p1 = pairs_list[:, 1].reshape(_B, _P, 1)
    b2 = jnp.reshape(b, (1, 1)).astype(jnp.float32)
    eps2 = jnp.reshape(epsilon, (1, 1)).astype(jnp.float32)

    x_flat = pooled_output.reshape(-1)
    p0_flat = pairs_list[:, 0]
    p1_flat = pairs_list[:, 1]
    w_flat = W.reshape(-1)
    be = jnp.concatenate([
        jnp.broadcast_to(b.astype(jnp.float32), (16,)),
        jnp.broadcast_to(jnp.asarray(epsilon, jnp.float32), (16,)),
    ])
    adj_sc = _sc_part(x_flat, p0_flat, p1_flat, w_flat, be).reshape(_BS, _S)

    adj_tc = _tc_part(x3, p0, p1, W, b2, eps2)

    adj = jnp.concatenate([adj_tc, adj_sc], axis=0)
    return adj.reshape(_B, _L, _L)


# speedup vs baseline: 2.2459x; 2.2459x over previous
import jax
import jax.numpy as jnp
from jax import lax
from jax.experimental import pallas as pl

_B = 512
_L = 16
_P = 240
_H = 768
_S = _L * _L
_PB = 8


def _fused_body(x_ref, p0_ref, p1_ref, w_ref, b_ref, eps_ref, out_ref):
    x = x_ref[...]
    w = w_ref[...]
    eps = eps_ref[0, 0]
    bias = b_ref[0, 0]
    logits = jnp.sum(x * w[None, :, :], axis=2, keepdims=True) + bias
    val = jax.nn.sigmoid(logits)
    idx = p0_ref[...] * _L + p1_ref[...]
    k3 = lax.broadcasted_iota(jnp.int32, (_PB, _P, 1), 1).astype(jnp.float32)
    packed = k3 + val
    lane = lax.broadcasted_iota(jnp.int32, (_PB, _P, _S), 2)
    comb = jnp.where(idx == lane, packed, -1.0)
    red = jnp.max(comb, axis=1)
    frac = red - jnp.floor(red)
    out_ref[...] = jnp.where(red >= 0.0, frac, 1.0) + eps


def kernel(pooled_output, pairs_list, passage_length, pairs_num, W, b, epsilon):
    del passage_length, pairs_num
    x3 = pooled_output.reshape(_B, _P, _H)
    p0 = pairs_list[:, 0].reshape(_B, _P, 1)
    p1 = pairs_list[:, 1].reshape(_B, _P, 1)
    b2 = jnp.reshape(b, (1, 1)).astype(jnp.float32)
    eps2 = jnp.reshape(epsilon, (1, 1)).astype(jnp.float32)

    grid = (_B // _PB,)
    adj = pl.pallas_call(
        _fused_body,
        grid=grid,
        in_specs=[
            pl.BlockSpec((_PB, _P, _H), lambda i: (i, 0, 0)),
            pl.BlockSpec((_PB, _P, 1), lambda i: (i, 0, 0)),
            pl.BlockSpec((_PB, _P, 1), lambda i: (i, 0, 0)),
            pl.BlockSpec((1, _H), lambda i: (0, 0)),
            pl.BlockSpec((1, 1), lambda i: (0, 0)),
            pl.BlockSpec((1, 1), lambda i: (0, 0)),
        ],
        out_specs=pl.BlockSpec((_PB, _S), lambda i: (i, 0)),
        out_shape=jax.ShapeDtypeStruct((_B, _S), jnp.float32),
    )(x3, p0, p1, W, b2, eps2)
    return adj.reshape(_B, _L, _L)
